# baseline (speedup 1.0000x reference)
import jax
import jax.numpy as jnp
from jax import lax
from jax.experimental import pallas as pl
from jax.experimental.pallas import tpu as pltpu
from jax.experimental.pallas import tpu_sc as plsc

N_NODES = 10000
IN_F = 128
H_F = 256
N_CLS = 40

NC, NS = 2, 16
NW = NC * NS
N_PAD = 10112
CHUNK = 128
ROWS_PER_TILE = N_PAD // NS


def _prep_body(feat_ref, norm_ref, out_ref):
    out_ref[...] = feat_ref[...] * norm_ref[...]


def _prep(features, norm2d):
    blk = 1000
    return pl.pallas_call(
        _prep_body,
        grid=(N_NODES // blk,),
        in_specs=[pl.BlockSpec((blk, IN_F), lambda i: (i, 0)),
                  pl.BlockSpec((blk, 1), lambda i: (i, 0))],
        out_specs=pl.BlockSpec((blk, IN_F), lambda i: (i, 0)),
        out_shape=jax.ShapeDtypeStruct((N_PAD, IN_F), jnp.float32),
    )(features, norm2d)


IDXB = 16


def _sc_body(fp_hbm, src_hbm, dst_hbm, norm_hbm, z2_hbm, z1_hbm,
             agg_out, s_out,
             srcb0, dstb0, srcb1, dstb1, nd0, rows0, nd1, rows1,
             agg_sp, s_sp, gsem0, nsem0, gsem1, nsem1):
    c = lax.axis_index("c")
    s = lax.axis_index("s")
    wid = s * NC + c

    idx_rows = src_hbm.shape[0]
    nchunk = idx_rows // NW
    nbatch = nchunk // IDXB
    irow0 = wid * nchunk

    r0 = s * ROWS_PER_TILE
    pltpu.sync_copy(z2_hbm, agg_sp.at[pl.ds(r0, ROWS_PER_TILE)])

    @pl.when(s == 0)
    def _():
        pltpu.sync_copy(z1_hbm, s_sp)

    plsc.subcore_barrier()

    srcb = (srcb0, srcb1)
    dstb = (dstb0, dstb1)
    ndv = (nd0, nd1)
    rows = (rows0, rows1)
    gsem = (gsem0, gsem1)
    nsem = (nsem0, nsem1)

    def idxload_batch(t):
        r = irow0 + t * IDXB
        pltpu.sync_copy(src_hbm.at[pl.ds(r, IDXB)], srcb[t % 2])
        pltpu.sync_copy(dst_hbm.at[pl.ds(r, IDXB)], dstb[t % 2])

    GSUB = 4
    SR = CHUNK // GSUB

    def fire(i):
        t, u = divmod(i, IDXB)
        b = i % 2
        for k in range(GSUB):
            pltpu.async_copy(
                fp_hbm.at[srcb[t % 2].at[u, pl.ds(k * SR, SR)]],
                rows[b].at[pl.ds(k * SR, SR)], gsem[b])
        pltpu.async_copy(norm_hbm.at[dstb[t % 2].at[u]], ndv[b], nsem[b])

    def drain(i):
        b = i % 2
        pltpu.make_async_copy(fp_hbm.at[pl.ds(0, CHUNK)], rows[b],
                              gsem[b]).wait()
        pltpu.make_async_copy(norm_hbm.at[pl.ds(0, CHUNK)], ndv[b],
                              nsem[b]).wait()

    def scat(i):
        t, u = divmod(i, IDXB)
        b = i % 2
        pltpu.sync_copy(rows[b], agg_sp.at[dstb[t % 2].at[u]], add=True)
        pltpu.sync_copy(ndv[b], s_sp.at[srcb[t % 2].at[u]], add=True)

    idxload_batch(0)
    fire(0)
    for i in range(nchunk):
        t, u = divmod(i, IDXB)
        if u == 1 and t + 1 < nbatch:
            idxload_batch(t + 1)
        if i + 1 < nchunk:
            fire(i + 1)
        drain(i)
        scat(i)

    plsc.subcore_barrier()

    pltpu.sync_copy(agg_sp.at[pl.ds(r0, ROWS_PER_TILE)],
                    agg_out.at[c, pl.ds(r0, ROWS_PER_TILE)])

    @pl.when(s == 0)
    def _():
        pltpu.sync_copy(s_sp, s_out.at[c])


def _sc_call(fp_ext, srcp, dstp, norm_ext, z2, z1):
    f = pl.kernel(
        _sc_body,
        out_type=(jax.ShapeDtypeStruct((NC, N_PAD, IN_F), jnp.float32),
                  jax.ShapeDtypeStruct((NC, N_PAD), jnp.float32)),
        mesh=plsc.VectorSubcoreMesh(core_axis_name="c", subcore_axis_name="s"),
        scratch_types=[
            pltpu.VMEM((IDXB, CHUNK), jnp.int32),
            pltpu.VMEM((IDXB, CHUNK), jnp.int32),
            pltpu.VMEM((IDXB, CHUNK), jnp.int32),
            pltpu.VMEM((IDXB, CHUNK), jnp.int32),
            pltpu.VMEM((CHUNK,), jnp.float32),
            pltpu.VMEM((CHUNK, IN_F), jnp.float32),
            pltpu.VMEM((CHUNK,), jnp.float32),
            pltpu.VMEM((CHUNK, IN_F), jnp.float32),
            pltpu.VMEM_SHARED((N_PAD, IN_F), jnp.float32),
            pltpu.VMEM_SHARED((N_PAD,), jnp.float32),
            pltpu.SemaphoreType.DMA,
            pltpu.SemaphoreType.DMA,
            pltpu.SemaphoreType.DMA,
            pltpu.SemaphoreType.DMA,
        ],
    )
    return f(fp_ext, srcp, dstp, norm_ext, z2, z1)


def _dense_body(agg0, agg1, norm_b, s0, s1, w1, b1r, w2, b2r, out_ref, h_acc):
    i = pl.program_id(0)

    @pl.when(i == 0)
    def _():
        h_acc[...] = jnp.zeros_like(h_acc)

    a = (agg0[0] + agg1[0]) * norm_b[...]
    x1 = jnp.dot(a, w1[...], preferred_element_type=jnp.float32) + b1r[...]
    x1 = jnp.maximum(x1, 0.0)
    cvec = norm_b[...] * (s0[...] + s1[...])
    h_acc[...] += jnp.sum(cvec * x1, axis=0, keepdims=True)

    @pl.when(i == pl.num_programs(0) - 1)
    def _():
        out_ref[...] = (jnp.dot(h_acc[...], w2[...],
                                preferred_element_type=jnp.float32)
                        * (1.0 / N_NODES) + b2r[...])


def _dense(aggp, norm2d, s0, s1, w1, b1r, w2, b2r):
    blk = 1000
    return pl.pallas_call(
        _dense_body,
        grid=(N_NODES // blk,),
        in_specs=[
            pl.BlockSpec((1, blk, IN_F), lambda i: (0, i, 0)),
            pl.BlockSpec((1, blk, IN_F), lambda i: (1, i, 0)),
            pl.BlockSpec((blk, 1), lambda i: (i, 0)),
            pl.BlockSpec((blk, 1), lambda i: (i, 0)),
            pl.BlockSpec((blk, 1), lambda i: (i, 0)),
            pl.BlockSpec((IN_F, H_F), lambda i: (0, 0)),
            pl.BlockSpec((1, H_F), lambda i: (0, 0)),
            pl.BlockSpec((H_F, N_CLS), lambda i: (0, 0)),
            pl.BlockSpec((1, N_CLS), lambda i: (0, 0)),
        ],
        out_specs=pl.BlockSpec((1, N_CLS), lambda i: (0, 0)),
        out_shape=jax.ShapeDtypeStruct((1, N_CLS), jnp.float32),
        scratch_shapes=[pltpu.VMEM((1, H_F), jnp.float32)],
    )(aggp, aggp, norm2d, s0, s1, w1, b1r, w2, b2r)


def kernel(features, edge_index, norm, W1, b1, W2, b2):
    n_edges = edge_index.shape[1]
    quantum = IDXB * CHUNK
    ept = -(-n_edges // (NW * quantum)) * quantum
    e_pad = ept * NW
    pad = e_pad - n_edges

    src = edge_index[0].astype(jnp.int32)
    dst = edge_index[1].astype(jnp.int32)
    srcp = jnp.concatenate(
        [src, jnp.full((pad,), N_NODES, jnp.int32)]).reshape(-1, CHUNK)
    dstp = jnp.concatenate(
        [dst, jnp.full((pad,), N_NODES, jnp.int32)]).reshape(-1, CHUNK)

    norm2d = norm[:, None]
    fp_ext = _prep(features, norm2d)
    norm_ext = jnp.concatenate(
        [norm, jnp.zeros((N_PAD - N_NODES,), jnp.float32)])

    z2 = jnp.zeros((ROWS_PER_TILE, IN_F), jnp.float32)
    z1 = jnp.zeros((N_PAD,), jnp.float32)

    aggp, sp = _sc_call(fp_ext, srcp, dstp, norm_ext, z2, z1)

    s0 = sp[0, :N_NODES, None]
    s1 = sp[1, :N_NODES, None]

    return _dense(aggp, norm2d, s0, s1,
                  W1, b1[None, :], W2, b2[None, :])


# speedup vs baseline: 1.0244x; 1.0244x over previous
import jax
import jax.numpy as jnp
from jax import lax
from jax.experimental import pallas as pl
from jax.experimental.pallas import tpu as pltpu
from jax.experimental.pallas import tpu_sc as plsc

N_NODES = 10000
IN_F = 128
H_F = 256
N_CLS = 40

NC, NS = 2, 16
NW = NC * NS
N_PAD = 10112
CHUNK = 128
ROWS_PER_TILE = N_PAD // NS


def _prep_body(feat_ref, norm_ref, out_ref):
    out_ref[...] = feat_ref[...] * norm_ref[...]


def _prep(features, norm2d):
    blk = 1000
    return pl.pallas_call(
        _prep_body,
        grid=(N_NODES // blk,),
        in_specs=[pl.BlockSpec((blk, IN_F), lambda i: (i, 0)),
                  pl.BlockSpec((blk, 1), lambda i: (i, 0))],
        out_specs=pl.BlockSpec((blk, IN_F), lambda i: (i, 0)),
        out_shape=jax.ShapeDtypeStruct((N_PAD, IN_F), jnp.float32),
    )(features, norm2d)


IDXB = 8
CF = 120
CS = 40


def _sc_body(fp_hbm, src_hbm, dst_hbm, norm_hbm, z2_hbm, z1_hbm,
             agg_out, s_out,
             srcb0, dstb0, srcb1, dstb1, nd0, rows0, nd1, rows1,
             agg_sp, s_sp, gsem0, nsem0, gsem1, nsem1):
    c = lax.axis_index("c")
    s = lax.axis_index("s")

    idx_rows = src_hbm.shape[0]
    scale = idx_rows // (NS * (CF + CS))

    r0 = s * ROWS_PER_TILE
    pltpu.sync_copy(z2_hbm, agg_sp.at[pl.ds(r0, ROWS_PER_TILE)])

    @pl.when(s == 0)
    def _():
        pltpu.sync_copy(z1_hbm, s_sp)

    plsc.subcore_barrier()

    srcb = (srcb0, srcb1)
    dstb = (dstb0, dstb1)
    ndv = (nd0, nd1)
    rows = (rows0, rows1)
    gsem = (gsem0, gsem1)
    nsem = (nsem0, nsem1)

    def run_edges(irow0, nchunk):
        nbatch = nchunk // IDXB

        def idxload_batch(t):
            r = irow0 + t * IDXB
            pltpu.sync_copy(src_hbm.at[pl.ds(r, IDXB)], srcb[t % 2])
            pltpu.sync_copy(dst_hbm.at[pl.ds(r, IDXB)], dstb[t % 2])

        def fire(i):
            t, u = divmod(i, IDXB)
            b = i % 2
            pltpu.async_copy(fp_hbm.at[srcb[t % 2].at[u]], rows[b], gsem[b])
            pltpu.async_copy(norm_hbm.at[dstb[t % 2].at[u]], ndv[b], nsem[b])

        def drain(i):
            b = i % 2
            pltpu.make_async_copy(fp_hbm.at[pl.ds(0, CHUNK)], rows[b],
                                  gsem[b]).wait()
            pltpu.make_async_copy(norm_hbm.at[pl.ds(0, CHUNK)], ndv[b],
                                  nsem[b]).wait()

        def scat(i):
            t, u = divmod(i, IDXB)
            b = i % 2
            pltpu.sync_copy(rows[b], agg_sp.at[dstb[t % 2].at[u]], add=True)
            pltpu.sync_copy(ndv[b], s_sp.at[srcb[t % 2].at[u]], add=True)

        idxload_batch(0)
        fire(0)
        for i in range(nchunk):
            t, u = divmod(i, IDXB)
            if u == 1 and t + 1 < nbatch:
                idxload_batch(t + 1)
            if i + 1 < nchunk:
                fire(i + 1)
            drain(i)
            scat(i)

    @pl.when(c == 0)
    def _():
        run_edges(s * (CF * scale), CF * scale)

    @pl.when(c == 1)
    def _():
        run_edges(NS * (CF * scale) + s * (CS * scale), CS * scale)

    plsc.subcore_barrier()

    pltpu.sync_copy(agg_sp.at[pl.ds(r0, ROWS_PER_TILE)],
                    agg_out.at[c, pl.ds(r0, ROWS_PER_TILE)])

    @pl.when(s == 0)
    def _():
        pltpu.sync_copy(s_sp, s_out.at[c])


def _sc_call(fp_ext, srcp, dstp, norm_ext, z2, z1):
    f = pl.kernel(
        _sc_body,
        out_type=(jax.ShapeDtypeStruct((NC, N_PAD, IN_F), jnp.float32),
                  jax.ShapeDtypeStruct((NC, N_PAD), jnp.float32)),
        mesh=plsc.VectorSubcoreMesh(core_axis_name="c", subcore_axis_name="s"),
        scratch_types=[
            pltpu.VMEM((IDXB, CHUNK), jnp.int32),
            pltpu.VMEM((IDXB, CHUNK), jnp.int32),
            pltpu.VMEM((IDXB, CHUNK), jnp.int32),
            pltpu.VMEM((IDXB, CHUNK), jnp.int32),
            pltpu.VMEM((CHUNK,), jnp.float32),
            pltpu.VMEM((CHUNK, IN_F), jnp.float32),
            pltpu.VMEM((CHUNK,), jnp.float32),
            pltpu.VMEM((CHUNK, IN_F), jnp.float32),
            pltpu.VMEM_SHARED((N_PAD, IN_F), jnp.float32),
            pltpu.VMEM_SHARED((N_PAD,), jnp.float32),
            pltpu.SemaphoreType.DMA,
            pltpu.SemaphoreType.DMA,
            pltpu.SemaphoreType.DMA,
            pltpu.SemaphoreType.DMA,
        ],
    )
    return f(fp_ext, srcp, dstp, norm_ext, z2, z1)


def _dense_body(agg0, agg1, norm_b, s0, s1, w1, b1r, w2, b2r, out_ref, h_acc):
    i = pl.program_id(0)

    @pl.when(i == 0)
    def _():
        h_acc[...] = jnp.zeros_like(h_acc)

    a = (agg0[0] + agg1[0]) * norm_b[...]
    x1 = jnp.dot(a, w1[...], preferred_element_type=jnp.float32) + b1r[...]
    x1 = jnp.maximum(x1, 0.0)
    cvec = norm_b[...] * (s0[...] + s1[...])
    h_acc[...] += jnp.sum(cvec * x1, axis=0, keepdims=True)

    @pl.when(i == pl.num_programs(0) - 1)
    def _():
        out_ref[...] = (jnp.dot(h_acc[...], w2[...],
                                preferred_element_type=jnp.float32)
                        * (1.0 / N_NODES) + b2r[...])


def _dense(aggp, norm2d, s0, s1, w1, b1r, w2, b2r):
    blk = 1000
    return pl.pallas_call(
        _dense_body,
        grid=(N_NODES // blk,),
        in_specs=[
            pl.BlockSpec((1, blk, IN_F), lambda i: (0, i, 0)),
            pl.BlockSpec((1, blk, IN_F), lambda i: (1, i, 0)),
            pl.BlockSpec((blk, 1), lambda i: (i, 0)),
            pl.BlockSpec((blk, 1), lambda i: (i, 0)),
            pl.BlockSpec((blk, 1), lambda i: (i, 0)),
            pl.BlockSpec((IN_F, H_F), lambda i: (0, 0)),
            pl.BlockSpec((1, H_F), lambda i: (0, 0)),
            pl.BlockSpec((H_F, N_CLS), lambda i: (0, 0)),
            pl.BlockSpec((1, N_CLS), lambda i: (0, 0)),
        ],
        out_specs=pl.BlockSpec((1, N_CLS), lambda i: (0, 0)),
        out_shape=jax.ShapeDtypeStruct((1, N_CLS), jnp.float32),
        scratch_shapes=[pltpu.VMEM((1, H_F), jnp.float32)],
    )(aggp, aggp, norm2d, s0, s1, w1, b1r, w2, b2r)


def kernel(features, edge_index, norm, W1, b1, W2, b2):
    n_edges = edge_index.shape[1]
    quantum = NS * (CF + CS) * CHUNK
    e_pad_total = -(-n_edges // quantum) * quantum
    ept = e_pad_total // NW
    e_pad = ept * NW
    pad = e_pad - n_edges

    src = edge_index[0].astype(jnp.int32)
    dst = edge_index[1].astype(jnp.int32)
    srcp = jnp.concatenate(
        [src, jnp.full((pad,), N_NODES, jnp.int32)]).reshape(-1, CHUNK)
    dstp = jnp.concatenate(
        [dst, jnp.full((pad,), N_NODES, jnp.int32)]).reshape(-1, CHUNK)

    norm2d = norm[:, None]
    fp_ext = _prep(features, norm2d)
    norm_ext = jnp.concatenate(
        [norm, jnp.zeros((N_PAD - N_NODES,), jnp.float32)])

    z2 = jnp.zeros((ROWS_PER_TILE, IN_F), jnp.float32)
    z1 = jnp.zeros((N_PAD,), jnp.float32)

    aggp, sp = _sc_call(fp_ext, srcp, dstp, norm_ext, z2, z1)

    s0 = sp[0, :N_NODES, None]
    s1 = sp[1, :N_NODES, None]

    return _dense(aggp, norm2d, s0, s1,
                  W1, b1[None, :], W2, b2[None, :])
